# Initial kernel scaffold; baseline (speedup 1.0000x reference)
import functools

import jax
import jax.numpy as jnp
from jax import lax
from jax.experimental import pallas as pl
from jax.experimental.pallas import tpu as pltpu
from jax.experimental.pallas import tpu_sc as plsc

N = 10000
E = 320000
D = 128
H = 16
C = 40

L = 16
NC = 2
NS = 16
NW = NC * NS
K = 128
T = -(-E // (NW * K))
E_PAD = T * NW * K
ACC_ROWS = T * K
RPS = ACC_ROWS // NS


def _spmm_sc(x, src3, dst3, zeros):
    mesh = plsc.VectorSubcoreMesh(
        core_axis_name="c", subcore_axis_name="s",
        num_cores=NC, num_subcores=NS)

    @functools.partial(
        pl.kernel,
        out_type=jax.ShapeDtypeStruct((NC, ACC_ROWS, L), jnp.float32),
        mesh=mesh,
        scratch_types=[
            pltpu.VMEM((T, K), jnp.int32),
            pltpu.VMEM((T, K), jnp.int32),
            pltpu.VMEM((K, L), jnp.float32),
            pltpu.VMEM_SHARED((ACC_ROWS, L), jnp.float32),
            pltpu.SemaphoreType.DMA,
        ],
    )
    def body(x_hbm, src_hbm, dst_hbm, z_hbm, out_hbm,
             src_v, dst_v, rows, acc, sem):
        cid = lax.axis_index("c")
        sid = lax.axis_index("s")
        wid = sid * NC + cid
        base = sid * RPS
        pltpu.sync_copy(z_hbm.at[pl.ds(base, RPS)], acc.at[pl.ds(base, RPS)])
        pltpu.sync_copy(src_hbm.at[wid], src_v)
        pltpu.sync_copy(dst_hbm.at[wid], dst_v)
        plsc.subcore_barrier()

        def step(t, carry):
            pltpu.async_copy(x_hbm.at[src_v.at[t]], rows, sem).wait()
            pltpu.sync_copy(rows, acc.at[dst_v.at[t]], add=True)
            return carry

        lax.fori_loop(0, T, step, 0)
        plsc.subcore_barrier()
        pltpu.sync_copy(acc.at[pl.ds(base, RPS)],
                        out_hbm.at[cid, pl.ds(base, RPS)])

    return body(x, src3, dst3, zeros)


def _mm1(features, W1):
    def body(f_ref, w_ref, o_ref):
        o_ref[...] = jnp.dot(f_ref[...], w_ref[...],
                             preferred_element_type=jnp.float32)
    return pl.pallas_call(
        body, out_shape=jax.ShapeDtypeStruct((N, H), jnp.float32),
    )(features, W1)


def _relu_sum(z):
    def body(z_ref, o_ref):
        o_ref[...] = jnp.maximum(z_ref[0] + z_ref[1], 0.0)
    return pl.pallas_call(
        body, out_shape=jax.ShapeDtypeStruct((ACC_ROWS, L), jnp.float32),
    )(z)


def _out_head(z, W2):
    def body(z_ref, w_ref, o_ref):
        q = jnp.dot(z_ref[0] + z_ref[1], w_ref[...],
                    preferred_element_type=jnp.float32)
        m = jnp.max(q, axis=1, keepdims=True)
        lse = m + jnp.log(jnp.sum(jnp.exp(q - m), axis=1, keepdims=True))
        o_ref[...] = (q - lse)[:N]
    return pl.pallas_call(
        body, out_shape=jax.ShapeDtypeStruct((N, C), jnp.float32),
    )(z, W2)


def kernel(features, edge_index, W1, W2):
    src = edge_index[0]
    dst = edge_index[1]
    pad = E_PAD - E
    src3 = jnp.concatenate([src, jnp.zeros((pad,), jnp.int32)]).reshape(NW, T, K)
    dst3 = jnp.concatenate([dst, jnp.full((pad,), N, jnp.int32)]).reshape(NW, T, K)
    zeros = jnp.zeros((ACC_ROWS, L), jnp.float32)

    y = _mm1(features, W1)
    z1 = _spmm_sc(y, src3, dst3, zeros)
    h = _relu_sum(z1)
    z2 = _spmm_sc(h, src3, dst3, zeros)
    return _out_head(z2, W2)


# speedup vs baseline: 13.5943x; 13.5943x over previous
import functools

import jax
import jax.numpy as jnp
from jax import lax
from jax.experimental import pallas as pl
from jax.experimental.pallas import tpu as pltpu
from jax.experimental.pallas import tpu_sc as plsc

N = 10000
E = 320000
D = 128
H = 16
C = 40

L = 16
NC = 2
NS = 16
NW = NC * NS
K = 128
T = -(-E // (NW * K))
E_PAD = T * NW * K
ACC_ROWS = T * K
RPS = ACC_ROWS // NS


def _spmm_sc(x, src3, dst3, zeros):
    mesh = plsc.VectorSubcoreMesh(
        core_axis_name="c", subcore_axis_name="s",
        num_cores=NC, num_subcores=NS)

    @functools.partial(
        pl.kernel,
        out_type=jax.ShapeDtypeStruct((NC, ACC_ROWS, L), jnp.float32),
        mesh=mesh,
        compiler_params=pltpu.CompilerParams(use_tc_tiling_on_sc=False),
        scratch_types=[
            pltpu.VMEM((T, K), jnp.int32),
            pltpu.VMEM((T, K), jnp.int32),
            pltpu.VMEM((K, L), jnp.float32),
            pltpu.VMEM_SHARED((ACC_ROWS, L), jnp.float32),
            pltpu.SemaphoreType.DMA,
        ],
    )
    def body(x_hbm, src_hbm, dst_hbm, z_hbm, out_hbm,
             src_v, dst_v, rows, acc, sem):
        cid = lax.axis_index("c")
        sid = lax.axis_index("s")
        wid = sid * NC + cid
        base = sid * RPS
        pltpu.sync_copy(z_hbm.at[pl.ds(base, RPS)], acc.at[pl.ds(base, RPS)])
        pltpu.sync_copy(src_hbm.at[wid], src_v)
        pltpu.sync_copy(dst_hbm.at[wid], dst_v)
        plsc.subcore_barrier()

        def step(t, carry):
            pltpu.async_copy(x_hbm.at[src_v.at[t]], rows, sem).wait()
            pltpu.sync_copy(rows, acc.at[dst_v.at[t]], add=True)
            return carry

        lax.fori_loop(0, T, step, 0)
        plsc.subcore_barrier()
        pltpu.sync_copy(acc.at[pl.ds(base, RPS)],
                        out_hbm.at[cid, pl.ds(base, RPS)])

    return body(x, src3, dst3, zeros)


def _mm1(features, W1):
    def body(f_ref, w_ref, o_ref):
        o_ref[...] = jnp.dot(f_ref[...], w_ref[...],
                             preferred_element_type=jnp.float32)
    return pl.pallas_call(
        body, out_shape=jax.ShapeDtypeStruct((N, H), jnp.float32),
    )(features, W1)


def _relu_sum(z):
    def body(z_ref, o_ref):
        o_ref[...] = jnp.maximum(z_ref[0] + z_ref[1], 0.0)
    return pl.pallas_call(
        body, out_shape=jax.ShapeDtypeStruct((ACC_ROWS, L), jnp.float32),
    )(z)


def _out_head(z, W2):
    def body(z_ref, w_ref, o_ref):
        q = jnp.dot(z_ref[0] + z_ref[1], w_ref[...],
                    preferred_element_type=jnp.float32)
        m = jnp.max(q, axis=1, keepdims=True)
        lse = m + jnp.log(jnp.sum(jnp.exp(q - m), axis=1, keepdims=True))
        o_ref[...] = (q - lse)[:N]
    return pl.pallas_call(
        body, out_shape=jax.ShapeDtypeStruct((N, C), jnp.float32),
    )(z, W2)


def kernel(features, edge_index, W1, W2):
    src = edge_index[0]
    dst = edge_index[1]
    pad = E_PAD - E
    src3 = jnp.concatenate([src, jnp.zeros((pad,), jnp.int32)]).reshape(NW, T, K)
    dst3 = jnp.concatenate([dst, jnp.full((pad,), N, jnp.int32)]).reshape(NW, T, K)
    zeros = jnp.zeros((ACC_ROWS, L), jnp.float32)

    y = _mm1(features, W1)
    z1 = _spmm_sc(y, src3, dst3, zeros)
    h = _relu_sum(z1)
    z2 = _spmm_sc(h, src3, dst3, zeros)
    return _out_head(z2, W2)


# speedup vs baseline: 16.7354x; 1.2311x over previous
import functools

import jax
import jax.numpy as jnp
from jax import lax
from jax.experimental import pallas as pl
from jax.experimental.pallas import tpu as pltpu
from jax.experimental.pallas import tpu_sc as plsc

N = 10000
E = 320000
D = 128
H = 16
C = 40

L = 16
NC = 2
NS = 16
NW = NC * NS
K = 128
NBUF = 4
T = NBUF * (-(-E // (NW * K * NBUF)))
E_PAD = T * NW * K
ACC_ROWS = 10112
RPS = ACC_ROWS // NS


def _spmm_sc(x, src3, dst3, zeros):
    mesh = plsc.VectorSubcoreMesh(
        core_axis_name="c", subcore_axis_name="s",
        num_cores=NC, num_subcores=NS)

    @functools.partial(
        pl.kernel,
        out_type=jax.ShapeDtypeStruct((NC, ACC_ROWS, L), jnp.float32),
        mesh=mesh,
        compiler_params=pltpu.CompilerParams(use_tc_tiling_on_sc=False),
        scratch_types=[
            pltpu.VMEM((T, K), jnp.int32),
            pltpu.VMEM((T, K), jnp.int32),
            pltpu.VMEM((NBUF, K, L), jnp.float32),
            pltpu.VMEM_SHARED((ACC_ROWS, L), jnp.float32),
            [pltpu.SemaphoreType.DMA] * NBUF,
        ],
    )
    def body(x_hbm, src_hbm, dst_hbm, z_hbm, out_hbm,
             src_v, dst_v, rows, acc, sems):
        cid = lax.axis_index("c")
        sid = lax.axis_index("s")
        wid = sid * NC + cid
        base = sid * RPS
        pltpu.sync_copy(z_hbm.at[pl.ds(base, RPS)], acc.at[pl.ds(base, RPS)])
        pltpu.sync_copy(src_hbm.at[wid], src_v)
        pltpu.sync_copy(dst_hbm.at[wid], dst_v)
        plsc.subcore_barrier()

        for b in range(NBUF - 1):
            pltpu.async_copy(x_hbm.at[src_v.at[b]], rows.at[b], sems[b])

        def outer(tg, carry):
            for b in range(NBUF):
                t = tg * NBUF + b
                nxt = t + NBUF - 1
                nb = (b - 1) % NBUF

                @pl.when(nxt < T)
                def _():
                    pltpu.async_copy(x_hbm.at[src_v.at[nxt]], rows.at[nb],
                                     sems[nb])
                pltpu.make_async_copy(x_hbm.at[src_v.at[t]], rows.at[b],
                                      sems[b]).wait()
                pltpu.sync_copy(rows.at[b], acc.at[dst_v.at[t]], add=True)
            return carry

        lax.fori_loop(0, T // NBUF, outer, 0)
        plsc.subcore_barrier()
        pltpu.sync_copy(acc.at[pl.ds(base, RPS)],
                        out_hbm.at[cid, pl.ds(base, RPS)])

    return body(x, src3, dst3, zeros)


def _mm1(features, W1):
    def body(f_ref, w_ref, o_ref):
        o_ref[...] = jnp.dot(f_ref[...], w_ref[...],
                             preferred_element_type=jnp.float32)
    return pl.pallas_call(
        body, out_shape=jax.ShapeDtypeStruct((N, H), jnp.float32),
    )(features, W1)


def _relu_sum(z):
    def body(z_ref, o_ref):
        o_ref[...] = jnp.maximum(z_ref[0] + z_ref[1], 0.0)
    return pl.pallas_call(
        body, out_shape=jax.ShapeDtypeStruct((ACC_ROWS, L), jnp.float32),
    )(z)


def _out_head(z, W2):
    def body(z_ref, w_ref, o_ref):
        q = jnp.dot(z_ref[0] + z_ref[1], w_ref[...],
                    preferred_element_type=jnp.float32)
        m = jnp.max(q, axis=1, keepdims=True)
        lse = m + jnp.log(jnp.sum(jnp.exp(q - m), axis=1, keepdims=True))
        o_ref[...] = (q - lse)[:N]
    return pl.pallas_call(
        body, out_shape=jax.ShapeDtypeStruct((N, C), jnp.float32),
    )(z, W2)


def kernel(features, edge_index, W1, W2):
    src = edge_index[0]
    dst = edge_index[1]
    pad = E_PAD - E
    src3 = jnp.concatenate([src, jnp.zeros((pad,), jnp.int32)]).reshape(NW, T, K)
    dst3 = jnp.concatenate([dst, jnp.full((pad,), N, jnp.int32)]).reshape(NW, T, K)
    zeros = jnp.zeros((ACC_ROWS, L), jnp.float32)

    y = _mm1(features, W1)
    z1 = _spmm_sc(y, src3, dst3, zeros)
    h = _relu_sum(z1)
    z2 = _spmm_sc(h, src3, dst3, zeros)
    return _out_head(z2, W2)
